# baseline (speedup 1.0000x reference)
import functools

import jax
import jax.numpy as jnp
from jax import lax
from jax.experimental import pallas as pl
from jax.experimental.pallas import tpu as pltpu
from jax.experimental.pallas import tpu_sc as plsc

B = 16384
MF_DIM = 64
MLP_DIM = 128
H1 = 128
H2 = 64

NC, NS = 2, 16
NW = NC * NS
BPW = B // NW
CHUNK = 128
NCHUNK = BPW // CHUNK


def _sc_gather(user, item, mf_u_t, mf_i_t, mlp_u_t, mlp_i_t):
    mesh = plsc.VectorSubcoreMesh(core_axis_name="c", subcore_axis_name="s")
    out_type = (
        jax.ShapeDtypeStruct((B, MF_DIM), jnp.float32),
        jax.ShapeDtypeStruct((B, MF_DIM), jnp.float32),
        jax.ShapeDtypeStruct((B, MLP_DIM), jnp.float32),
        jax.ShapeDtypeStruct((B, MLP_DIM), jnp.float32),
    )

    @functools.partial(
        pl.kernel,
        mesh=mesh,
        out_type=out_type,
        compiler_params=pltpu.CompilerParams(use_tc_tiling_on_sc=False),
        scratch_types=[
            pltpu.VMEM((BPW,), jnp.int32),
            pltpu.VMEM((BPW,), jnp.int32),
            pltpu.VMEM((CHUNK, MF_DIM), jnp.float32),
            pltpu.VMEM((CHUNK, MF_DIM), jnp.float32),
            pltpu.VMEM((CHUNK, MLP_DIM), jnp.float32),
            pltpu.VMEM((CHUNK, MLP_DIM), jnp.float32),
            pltpu.SemaphoreType.DMA,
        ],
    )
    def k(mf_u_hbm, mf_i_hbm, mlp_u_hbm, mlp_i_hbm, user_hbm, item_hbm,
          o_mf_u, o_mf_i, o_mlp_u, o_mlp_i,
          uidx, iidx, b_mf_u, b_mf_i, b_mlp_u, b_mlp_i, sem):
        wid = lax.axis_index("s") * NC + lax.axis_index("c")
        base = wid * BPW
        pltpu.sync_copy(user_hbm.at[pl.ds(base, BPW)], uidx)
        pltpu.sync_copy(item_hbm.at[pl.ds(base, BPW)], iidx)

        @pl.loop(0, NCHUNK)
        def _(ci):
            off = ci * CHUNK
            u_sl = uidx.at[pl.ds(off, CHUNK)]
            i_sl = iidx.at[pl.ds(off, CHUNK)]
            c1 = pltpu.async_copy(mf_u_hbm.at[u_sl], b_mf_u, sem)
            c2 = pltpu.async_copy(mf_i_hbm.at[i_sl], b_mf_i, sem)
            c3 = pltpu.async_copy(mlp_u_hbm.at[u_sl], b_mlp_u, sem)
            c4 = pltpu.async_copy(mlp_i_hbm.at[i_sl], b_mlp_i, sem)
            c1.wait()
            c2.wait()
            c3.wait()
            c4.wait()
            dst = pl.ds(base + off, CHUNK)
            pltpu.sync_copy(b_mf_u, o_mf_u.at[dst])
            pltpu.sync_copy(b_mf_i, o_mf_i.at[dst])
            pltpu.sync_copy(b_mlp_u, o_mlp_u.at[dst])
            pltpu.sync_copy(b_mlp_i, o_mlp_i.at[dst])

    return k(mf_u_t, mf_i_t, mlp_u_t, mlp_i_t, user, item)


BM = 2048


def _tc_body(mf_u_ref, mf_i_ref, mlp_u_ref, mlp_i_ref,
             w1a_ref, w1b_ref, b1_ref, w2_ref, b2_ref,
             wp_mf_ref, wp_mlp_ref, bp_ref, o_ref):
    hp = jax.lax.Precision.HIGHEST
    h1 = jnp.dot(mlp_u_ref[...], w1a_ref[...], precision=hp,
                 preferred_element_type=jnp.float32)
    h1 = h1 + jnp.dot(mlp_i_ref[...], w1b_ref[...], precision=hp,
                      preferred_element_type=jnp.float32)
    h1 = jnp.maximum(h1 + b1_ref[...], 0.0)
    h2 = jnp.dot(h1, w2_ref[...], precision=hp,
                 preferred_element_type=jnp.float32)
    h2 = jnp.maximum(h2 + b2_ref[...], 0.0)
    mf = mf_u_ref[...] * mf_i_ref[...]
    logit = jnp.sum(mf * wp_mf_ref[...], axis=1, keepdims=True)
    logit = logit + jnp.sum(h2 * wp_mlp_ref[...], axis=1, keepdims=True)
    o_ref[...] = jax.nn.sigmoid(logit + bp_ref[...])


def _tc_mlp(mf_u, mf_i, mlp_u, mlp_i, W1, b1, W2, b2, Wp, bp):
    w1a = W1[:MLP_DIM]
    w1b = W1[MLP_DIM:]
    wp_mf = Wp[:MF_DIM].reshape(1, MF_DIM)
    wp_mlp = Wp[MF_DIM:].reshape(1, H2)
    b1r = b1.reshape(1, H1)
    b2r = b2.reshape(1, H2)
    bpr = bp.reshape(1, 1)

    grid = (B // BM,)
    row = lambda i: (i, 0)
    rep = lambda i: (0, 0)
    out = pl.pallas_call(
        _tc_body,
        grid=grid,
        in_specs=[
            pl.BlockSpec((BM, MF_DIM), row),
            pl.BlockSpec((BM, MF_DIM), row),
            pl.BlockSpec((BM, MLP_DIM), row),
            pl.BlockSpec((BM, MLP_DIM), row),
            pl.BlockSpec((MLP_DIM, H1), rep),
            pl.BlockSpec((MLP_DIM, H1), rep),
            pl.BlockSpec((1, H1), rep),
            pl.BlockSpec((H1, H2), rep),
            pl.BlockSpec((1, H2), rep),
            pl.BlockSpec((1, MF_DIM), rep),
            pl.BlockSpec((1, H2), rep),
            pl.BlockSpec((1, 1), rep),
        ],
        out_specs=pl.BlockSpec((BM, 1), row),
        out_shape=jax.ShapeDtypeStruct((B, 1), jnp.float32),
        compiler_params=pltpu.CompilerParams(
            dimension_semantics=("arbitrary",),
        ),
    )(mf_u, mf_i, mlp_u, mlp_i, w1a, w1b, b1r, W2, b2r,
      wp_mf, wp_mlp, bpr)
    return out.reshape(-1)


def kernel(user, item, mf_user_table, mf_item_table, mlp_user_table,
           mlp_item_table, W1, b1, W2, b2, Wp, bp):
    user = user.astype(jnp.int32)
    item = item.astype(jnp.int32)
    mf_u, mf_i, mlp_u, mlp_i = _sc_gather(
        user, item, mf_user_table, mf_item_table, mlp_user_table,
        mlp_item_table)
    return _tc_mlp(mf_u, mf_i, mlp_u, mlp_i, W1, b1, W2, b2, Wp, bp)


# speedup vs baseline: 1.0223x; 1.0223x over previous
import functools

import jax
import jax.numpy as jnp
from jax import lax
from jax.experimental import pallas as pl
from jax.experimental.pallas import tpu as pltpu
from jax.experimental.pallas import tpu_sc as plsc

B = 16384
MF_DIM = 64
MLP_DIM = 128
H1 = 128
H2 = 64

NC, NS = 2, 16
NW = NC * NS
BPW = B // NW
CHUNK = 64
NCHUNK = BPW // CHUNK
NB = 2


def _sc_gather4(user, item, umf, imf, mf_u2, mf_i2, mlp_u_t, mlp_i_t):
    mesh = plsc.VectorSubcoreMesh(core_axis_name="c", subcore_axis_name="s")
    out_type = (
        jax.ShapeDtypeStruct((B, 128), jnp.float32),
        jax.ShapeDtypeStruct((B, 128), jnp.float32),
        jax.ShapeDtypeStruct((B, MLP_DIM), jnp.float32),
        jax.ShapeDtypeStruct((B, MLP_DIM), jnp.float32),
    )

    @functools.partial(
        pl.kernel,
        mesh=mesh,
        out_type=out_type,
        scratch_types=[
            pltpu.VMEM((BPW,), jnp.int32),
            pltpu.VMEM((BPW,), jnp.int32),
            pltpu.VMEM((BPW,), jnp.int32),
            pltpu.VMEM((BPW,), jnp.int32),
            pltpu.VMEM((NB, CHUNK, 128), jnp.float32),
            pltpu.VMEM((NB, CHUNK, 128), jnp.float32),
            pltpu.VMEM((NB, CHUNK, MLP_DIM), jnp.float32),
            pltpu.VMEM((NB, CHUNK, MLP_DIM), jnp.float32),
            pltpu.SemaphoreType.DMA((NB,)),
            pltpu.SemaphoreType.DMA((NB,)),
        ],
    )
    def k(mf_u_hbm, mf_i_hbm, mlp_u_hbm, mlp_i_hbm,
          user_hbm, item_hbm, umf_hbm, imf_hbm,
          o_mf_u, o_mf_i, o_mlp_u, o_mlp_i,
          uidx, iidx, umidx, imidx, b_mfu, b_mfi, b_mlu, b_mli,
          gsem, osem):
        wid = lax.axis_index("s") * NC + lax.axis_index("c")
        base = wid * BPW
        pltpu.sync_copy(user_hbm.at[pl.ds(base, BPW)], uidx)
        pltpu.sync_copy(item_hbm.at[pl.ds(base, BPW)], iidx)
        pltpu.sync_copy(umf_hbm.at[pl.ds(base, BPW)], umidx)
        pltpu.sync_copy(imf_hbm.at[pl.ds(base, BPW)], imidx)

        def g_copies(cur, b, make_only=False):
            off = cur * CHUNK
            mk = pltpu.make_async_copy if make_only else pltpu.async_copy
            c = [
                mk(mf_u_hbm.at[umidx.at[pl.ds(off, CHUNK)]], b_mfu.at[b],
                   gsem.at[b]),
                mk(mf_i_hbm.at[imidx.at[pl.ds(off, CHUNK)]], b_mfi.at[b],
                   gsem.at[b]),
                mk(mlp_u_hbm.at[uidx.at[pl.ds(off, CHUNK)]], b_mlu.at[b],
                   gsem.at[b]),
                mk(mlp_i_hbm.at[iidx.at[pl.ds(off, CHUNK)]], b_mli.at[b],
                   gsem.at[b]),
            ]
            return c

        def o_copies(cur, b, make_only=False):
            dst = pl.ds(base + cur * CHUNK, CHUNK)
            mk = pltpu.make_async_copy if make_only else pltpu.async_copy
            return [
                mk(b_mfu.at[b], o_mf_u.at[dst], osem.at[b]),
                mk(b_mfi.at[b], o_mf_i.at[dst], osem.at[b]),
                mk(b_mlu.at[b], o_mlp_u.at[dst], osem.at[b]),
                mk(b_mli.at[b], o_mlp_i.at[dst], osem.at[b]),
            ]

        for b in range(NB):
            g_copies(b, b)

        @pl.loop(0, NCHUNK, step=NB)
        def _(ci):
            for b in range(NB):
                cur = ci + b
                for c in g_copies(cur, b, make_only=True):
                    c.wait()
                o_copies(cur, b)
                nxt = cur + NB

                @pl.when(nxt < NCHUNK)
                def _():
                    for c in o_copies(cur, b, make_only=True):
                        c.wait()
                    g_copies(nxt, b)

        for b in range(NB):
            for c in o_copies(NCHUNK - NB + b, b, make_only=True):
                c.wait()

    return k(mf_u2, mf_i2, mlp_u_t, mlp_i_t, user, item, umf, imf)


BM = 2048


def _tc_body(mfu_ref, mfi_ref, mlp_u_ref, mlp_i_ref, uh_ref, ih_ref,
             w1a_ref, w1b_ref, b1_ref, w2_ref, b2_ref,
             wp_mf_ref, wp_mlp_ref, bp_ref, o_ref):
    hp = jax.lax.Precision.HIGHEST
    h1 = jnp.dot(mlp_u_ref[...], w1a_ref[...], precision=hp,
                 preferred_element_type=jnp.float32)
    h1 = h1 + jnp.dot(mlp_i_ref[...], w1b_ref[...], precision=hp,
                      preferred_element_type=jnp.float32)
    h1 = jnp.maximum(h1 + b1_ref[...], 0.0)
    h2 = jnp.dot(h1, w2_ref[...], precision=hp,
                 preferred_element_type=jnp.float32)
    h2 = jnp.maximum(h2 + b2_ref[...], 0.0)
    g_u = mfu_ref[...]
    g_i = mfi_ref[...]
    mf_u = jnp.where(uh_ref[...] == 0, g_u[:, :MF_DIM], g_u[:, MF_DIM:])
    mf_i = jnp.where(ih_ref[...] == 0, g_i[:, :MF_DIM], g_i[:, MF_DIM:])
    mf = mf_u * mf_i
    logit = jnp.sum(mf * wp_mf_ref[...], axis=1, keepdims=True)
    logit = logit + jnp.sum(h2 * wp_mlp_ref[...], axis=1, keepdims=True)
    o_ref[...] = jax.nn.sigmoid(logit + bp_ref[...])


def _tc_mlp(mf_u_g, mf_i_g, mlp_u, mlp_i, uh, ih, W1, b1, W2, b2, Wp, bp):
    w1a = W1[:MLP_DIM]
    w1b = W1[MLP_DIM:]
    wp_mf = Wp[:MF_DIM].reshape(1, MF_DIM)
    wp_mlp = Wp[MF_DIM:].reshape(1, H2)
    b1r = b1.reshape(1, H1)
    b2r = b2.reshape(1, H2)
    bpr = bp.reshape(1, 1)

    grid = (B // BM,)
    row = lambda i: (i, 0)
    rep = lambda i: (0, 0)
    out = pl.pallas_call(
        _tc_body,
        grid=grid,
        in_specs=[
            pl.BlockSpec((BM, 128), row),
            pl.BlockSpec((BM, 128), row),
            pl.BlockSpec((BM, MLP_DIM), row),
            pl.BlockSpec((BM, MLP_DIM), row),
            pl.BlockSpec((BM, 1), row),
            pl.BlockSpec((BM, 1), row),
            pl.BlockSpec((MLP_DIM, H1), rep),
            pl.BlockSpec((MLP_DIM, H1), rep),
            pl.BlockSpec((1, H1), rep),
            pl.BlockSpec((H1, H2), rep),
            pl.BlockSpec((1, H2), rep),
            pl.BlockSpec((1, MF_DIM), rep),
            pl.BlockSpec((1, H2), rep),
            pl.BlockSpec((1, 1), rep),
        ],
        out_specs=pl.BlockSpec((BM, 1), row),
        out_shape=jax.ShapeDtypeStruct((B, 1), jnp.float32),
        compiler_params=pltpu.CompilerParams(
            dimension_semantics=("arbitrary",),
        ),
    )(mf_u_g, mf_i_g, mlp_u, mlp_i, uh, ih, w1a, w1b, b1r, W2, b2r,
      wp_mf, wp_mlp, bpr)
    return out.reshape(-1)


def kernel(user, item, mf_user_table, mf_item_table, mlp_user_table,
           mlp_item_table, W1, b1, W2, b2, Wp, bp):
    user = user.astype(jnp.int32)
    item = item.astype(jnp.int32)
    umf = jax.lax.shift_right_logical(user, 1)
    imf = jax.lax.shift_right_logical(item, 1)
    uh = (user & 1).reshape(B, 1)
    ih = (item & 1).reshape(B, 1)
    nu = mf_user_table.shape[0]
    ni = mf_item_table.shape[0]
    mf_u2 = mf_user_table.reshape(nu // 2, 2 * MF_DIM)
    mf_i2 = mf_item_table.reshape(ni // 2, 2 * MF_DIM)
    mf_u_g, mf_i_g, mlp_u, mlp_i = _sc_gather4(
        user, item, umf, imf, mf_u2, mf_i2, mlp_user_table, mlp_item_table)
    return _tc_mlp(mf_u_g, mf_i_g, mlp_u, mlp_i, uh, ih,
                   W1, b1, W2, b2, Wp, bp)
